# baseline (speedup 1.0000x reference)
import jax
import jax.numpy as jnp
import numpy as np
from jax import lax
from jax.experimental import pallas as pl
from jax.experimental.pallas import tpu as pltpu
from jax.experimental.pallas import tpu_sc as plsc

_B, _A, _F = 1024, 64, 128
_D, _FB, _CW = 6, 4, 32
_G = 16
_GA = _G * _A

_NW = 32
_SPW = _B // _NW
_FC = _F // 16

_DN0 = (((0,), (0,)), ((), ()))


_NH = 2
_CH = _B // _NH
_SPWC = _CH // _NW


def _sc_body(h, atoms_hbm, eloc_hbm, out_hbm,
             tv0, tv1, ev0, ev1, ac0, ac1, sin0, sin1, sout0, sout1):
    c = lax.axis_index("c")
    s = lax.axis_index("s")
    wid = s * 2 + c
    base = h * _CH + wid * _SPWC
    obase = wid * _SPWC
    tvs, evs, acs = (tv0, tv1), (ev0, ev1), (ac0, ac1)
    sins, souts = (sin0, sin1), (sout0, sout1)

    for tv in tvs:
        for j in range(_FC):
            tv[_A, pl.ds(j * 16, 16)] = jnp.zeros((16,), jnp.float32)

    def in_copies(g, bi):
        return (
            pltpu.make_async_copy(atoms_hbm.at[pl.ds(g * _A, _A)],
                                  tvs[bi].at[pl.ds(0, _A)], sins[bi]),
            pltpu.make_async_copy(eloc_hbm.at[g],
                                  evs[bi].at[pl.ds(0, _A * _D)], sins[bi]),
        )

    def out_copy(g, bi):
        return pltpu.make_async_copy(
            acs[bi], out_hbm.at[pl.ds((g - base + obase) * _A, _A)],
            souts[bi])

    for cp in in_copies(base, 0):
        cp.start()

    def pair_body(k, carry):
        for bi in range(2):
            g = base + k * 2 + bi
            for cp in in_copies(g, bi):
                cp.wait()

            @pl.when(g + 1 < base + _SPWC)
            def _():
                for cp in in_copies(g + 1, 1 - bi):
                    cp.start()

            @pl.when(g - 2 >= base)
            def _():
                out_copy(g - 2, bi).wait()

            tv, av = tvs[bi], acs[bi]

            def atom_body(a, carry2):
                evec = evs[bi][pl.ds(a * _D, 16)]
                es = [evec[d] for d in range(_D)]
                for j in range(_FC):
                    sl = pl.ds(j * 16, 16)
                    r = [tv[a, sl]] + [tv[e, sl] for e in es]
                    av[a, sl] = ((r[0] + r[1]) + (r[2] + r[3])) + \
                                ((r[4] + r[5]) + r[6])
                return carry2

            lax.fori_loop(0, _A, atom_body, 0, unroll=4)
            out_copy(g, bi).start()
        return carry

    lax.fori_loop(0, _SPWC // 2, pair_body, 0)
    out_copy(base + _SPWC - 2, 0).wait()
    out_copy(base + _SPWC - 1, 1).wait()


def _tc_body(et_ref, sa_ref, bt_ref, wa_ref, wb_ref, bias_ref,
             sel_ref, red_ref, out_ref):
    et = et_ref[...]
    sa16 = sa_ref[...].astype(jnp.bfloat16)
    bt16 = bt_ref[...].astype(jnp.bfloat16)

    valid16 = (et != -1).astype(jnp.bfloat16)
    ones_col = jnp.ones((_D, 1), jnp.bfloat16)
    deg = lax.dot_general(valid16, ones_col, _DN0,
                          preferred_element_type=jnp.float32)
    mask = (deg.astype(jnp.bfloat16) == sel_ref[...]).astype(jnp.bfloat16)

    y = jnp.dot(sa16, wa_ref[...], preferred_element_type=jnp.float32)
    y = y + lax.dot_general(bt16, wb_ref[...], _DN0,
                            preferred_element_type=jnp.float32)
    y = y + bias_ref[...]
    ym = y.astype(jnp.bfloat16) * mask
    out_ref[...] = jnp.dot(ym, red_ref[...], preferred_element_type=jnp.float32)


def kernel(atoms, bonds, edges, W, b):
    atoms2 = atoms.reshape(_B * _A, _F)
    eloc = jnp.where(edges >= 0, edges, _A).reshape(_B, _A * _D)
    et = edges.reshape(_B * _A, _D).T
    bt = bonds.reshape(_B * _A, _D * _FB).T
    wa = W[:, :_F, :].transpose(1, 0, 2).reshape(_F, _D * _CW
                                                 ).astype(jnp.bfloat16)
    wb = jnp.tile(W[:, _F:, :].transpose(1, 0, 2).reshape(_FB, _D * _CW),
                  (_D, 1)).astype(jnp.bfloat16)
    bias = b.reshape(1, _D * _CW)
    sel = jnp.asarray(np.repeat(np.arange(_D, dtype=np.float32), _CW)
                      ).reshape(1, _D * _CW).astype(jnp.bfloat16)
    red = jnp.asarray(
        (np.arange(_D * _CW)[:, None] % _CW == np.arange(_CW)[None, :])
        .astype(np.float32)).astype(jnp.bfloat16)

    import functools
    halves = []
    for h in range(_NH):
        sc_gather = pl.kernel(
            functools.partial(_sc_body, h),
            out_type=jax.ShapeDtypeStruct((_CH * _A, _F), jnp.float32),
            mesh=plsc.VectorSubcoreMesh(core_axis_name="c",
                                        subcore_axis_name="s"),
            scratch_types=[
                pltpu.VMEM((_A + 1, _F), jnp.float32),
                pltpu.VMEM((_A + 1, _F), jnp.float32),
                pltpu.VMEM((_A * 8 + 16,), jnp.int32),
                pltpu.VMEM((_A * 8 + 16,), jnp.int32),
                pltpu.VMEM((_A, _F), jnp.float32),
                pltpu.VMEM((_A, _F), jnp.float32),
                pltpu.SemaphoreType.DMA,
                pltpu.SemaphoreType.DMA,
                pltpu.SemaphoreType.DMA,
                pltpu.SemaphoreType.DMA,
            ],
            name=f"sc_gather_h{h}",
        )
        summed = sc_gather(atoms2, eloc)

        hb = _CH // _G
        out_h = pl.pallas_call(
            _tc_body,
            grid=(hb,),
            in_specs=[
                pl.BlockSpec((_D, _GA), lambda i, h=h: (0, i + h * hb)),
                pl.BlockSpec((_GA, _F), lambda i: (i, 0)),
                pl.BlockSpec((_D * _FB, _GA), lambda i, h=h: (0, i + h * hb)),
                pl.BlockSpec((_F, _D * _CW), lambda i: (0, 0)),
                pl.BlockSpec((_D * _FB, _D * _CW), lambda i: (0, 0)),
                pl.BlockSpec((1, _D * _CW), lambda i: (0, 0)),
                pl.BlockSpec((1, _D * _CW), lambda i: (0, 0)),
                pl.BlockSpec((_D * _CW, _CW), lambda i: (0, 0)),
            ],
            out_specs=pl.BlockSpec((_GA, _CW), lambda i: (i, 0)),
            out_shape=jax.ShapeDtypeStruct((_CH * _A, _CW), jnp.float32),
        )(et, summed, bt, wa, wb, bias, sel, red)
        halves.append(out_h)

    out = jnp.concatenate(halves, axis=0)
    return out.reshape(_B, _A, _CW)


# speedup vs baseline: 1.0068x; 1.0068x over previous
import jax
import jax.numpy as jnp
import numpy as np
from jax import lax
from jax.experimental import pallas as pl
from jax.experimental.pallas import tpu as pltpu
from jax.experimental.pallas import tpu_sc as plsc

_B, _A, _F = 1024, 64, 128
_D, _FB, _CW = 6, 4, 32
_G = 16
_GA = _G * _A

_NW = 32
_SPW = _B // _NW
_FC = _F // 16

_DN0 = (((0,), (0,)), ((), ()))


_NH = 2
_CH = _B // _NH
_SPWC = _CH // _NW


def _sc_body(h, atoms_hbm, eloc_hbm, out_hbm,
             tv0, tv1, ev0, ev1, ac0, ac1, sin0, sin1, sout0, sout1):
    c = lax.axis_index("c")
    s = lax.axis_index("s")
    wid = s * 2 + c
    base = h * _CH + wid * _SPWC
    obase = wid * _SPWC
    tvs, evs, acs = (tv0, tv1), (ev0, ev1), (ac0, ac1)
    sins, souts = (sin0, sin1), (sout0, sout1)

    for tv in tvs:
        for j in range(_FC):
            tv[_A, pl.ds(j * 16, 16)] = jnp.zeros((16,), jnp.float32)

    def in_copies(g, bi):
        return (
            pltpu.make_async_copy(atoms_hbm.at[pl.ds(g * _A, _A)],
                                  tvs[bi].at[pl.ds(0, _A)], sins[bi]),
            pltpu.make_async_copy(eloc_hbm.at[g],
                                  evs[bi].at[pl.ds(0, _A * _D)], sins[bi]),
        )

    def out_copy(g, bi):
        return pltpu.make_async_copy(
            acs[bi], out_hbm.at[pl.ds((g - base + obase) * _A, _A)],
            souts[bi])

    for cp in in_copies(base, 0):
        cp.start()

    def pair_body(k, carry):
        for bi in range(2):
            g = base + k * 2 + bi
            for cp in in_copies(g, bi):
                cp.wait()

            @pl.when(g + 1 < base + _SPWC)
            def _():
                for cp in in_copies(g + 1, 1 - bi):
                    cp.start()

            @pl.when(g - 2 >= base)
            def _():
                out_copy(g - 2, bi).wait()

            tv, av = tvs[bi], acs[bi]

            def atom_body(a, carry2):
                evec = evs[bi][pl.ds(a * _D, 16)]
                es = [evec[d] for d in range(_D)]
                for j in range(_FC):
                    sl = pl.ds(j * 16, 16)
                    r = [tv[a, sl]] + [tv[e, sl] for e in es]
                    av[a, sl] = ((r[0] + r[1]) + (r[2] + r[3])) + \
                                ((r[4] + r[5]) + r[6])
                return carry2

            lax.fori_loop(0, _A, atom_body, 0, unroll=2)
            out_copy(g, bi).start()
        return carry

    lax.fori_loop(0, _SPWC // 2, pair_body, 0)
    out_copy(base + _SPWC - 2, 0).wait()
    out_copy(base + _SPWC - 1, 1).wait()


def _tc_body(et_ref, sa_ref, bt_ref, wa_ref, wb_ref, bias_ref,
             sel_ref, red_ref, out_ref):
    et = et_ref[...]
    sa16 = sa_ref[...].astype(jnp.bfloat16)
    bt16 = bt_ref[...].astype(jnp.bfloat16)

    valid16 = (et != -1).astype(jnp.bfloat16)
    ones_col = jnp.ones((_D, 1), jnp.bfloat16)
    deg = lax.dot_general(valid16, ones_col, _DN0,
                          preferred_element_type=jnp.float32)
    mask = (deg.astype(jnp.bfloat16) == sel_ref[...]).astype(jnp.bfloat16)

    y = jnp.dot(sa16, wa_ref[...], preferred_element_type=jnp.float32)
    y = y + lax.dot_general(bt16, wb_ref[...], _DN0,
                            preferred_element_type=jnp.float32)
    y = y + bias_ref[...]
    ym = y.astype(jnp.bfloat16) * mask
    out_ref[...] = jnp.dot(ym, red_ref[...], preferred_element_type=jnp.float32)


def kernel(atoms, bonds, edges, W, b):
    atoms2 = atoms.reshape(_B * _A, _F)
    eloc = jnp.where(edges >= 0, edges, _A).reshape(_B, _A * _D)
    et = edges.reshape(_B * _A, _D).T
    bt = bonds.reshape(_B * _A, _D * _FB).T
    wa = W[:, :_F, :].transpose(1, 0, 2).reshape(_F, _D * _CW
                                                 ).astype(jnp.bfloat16)
    wb = jnp.tile(W[:, _F:, :].transpose(1, 0, 2).reshape(_FB, _D * _CW),
                  (_D, 1)).astype(jnp.bfloat16)
    bias = b.reshape(1, _D * _CW)
    sel = jnp.asarray(np.repeat(np.arange(_D, dtype=np.float32), _CW)
                      ).reshape(1, _D * _CW).astype(jnp.bfloat16)
    red = jnp.asarray(
        (np.arange(_D * _CW)[:, None] % _CW == np.arange(_CW)[None, :])
        .astype(np.float32)).astype(jnp.bfloat16)

    import functools
    halves = []
    for h in range(_NH):
        sc_gather = pl.kernel(
            functools.partial(_sc_body, h),
            out_type=jax.ShapeDtypeStruct((_CH * _A, _F), jnp.float32),
            mesh=plsc.VectorSubcoreMesh(core_axis_name="c",
                                        subcore_axis_name="s"),
            scratch_types=[
                pltpu.VMEM((_A + 1, _F), jnp.float32),
                pltpu.VMEM((_A + 1, _F), jnp.float32),
                pltpu.VMEM((_A * 8 + 16,), jnp.int32),
                pltpu.VMEM((_A * 8 + 16,), jnp.int32),
                pltpu.VMEM((_A, _F), jnp.float32),
                pltpu.VMEM((_A, _F), jnp.float32),
                pltpu.SemaphoreType.DMA,
                pltpu.SemaphoreType.DMA,
                pltpu.SemaphoreType.DMA,
                pltpu.SemaphoreType.DMA,
            ],
            name=f"sc_gather_h{h}",
        )
        summed = sc_gather(atoms2, eloc)

        hb = _CH // _G
        out_h = pl.pallas_call(
            _tc_body,
            grid=(hb,),
            in_specs=[
                pl.BlockSpec((_D, _GA), lambda i, h=h: (0, i + h * hb)),
                pl.BlockSpec((_GA, _F), lambda i: (i, 0)),
                pl.BlockSpec((_D * _FB, _GA), lambda i, h=h: (0, i + h * hb)),
                pl.BlockSpec((_F, _D * _CW), lambda i: (0, 0)),
                pl.BlockSpec((_D * _FB, _D * _CW), lambda i: (0, 0)),
                pl.BlockSpec((1, _D * _CW), lambda i: (0, 0)),
                pl.BlockSpec((1, _D * _CW), lambda i: (0, 0)),
                pl.BlockSpec((_D * _CW, _CW), lambda i: (0, 0)),
            ],
            out_specs=pl.BlockSpec((_GA, _CW), lambda i: (i, 0)),
            out_shape=jax.ShapeDtypeStruct((_CH * _A, _CW), jnp.float32),
        )(et, summed, bt, wa, wb, bias, sel, red)
        halves.append(out_h)

    out = jnp.concatenate(halves, axis=0)
    return out.reshape(_B, _A, _CW)


# speedup vs baseline: 1.0418x; 1.0347x over previous
import jax
import jax.numpy as jnp
import numpy as np
from jax import lax
from jax.experimental import pallas as pl
from jax.experimental.pallas import tpu as pltpu
from jax.experimental.pallas import tpu_sc as plsc

_B, _A, _F = 1024, 64, 128
_D, _FB, _CW = 6, 4, 32
_G = 16
_GA = _G * _A

_NW = 32
_SPW = _B // _NW
_FC = _F // 16

_DN0 = (((0,), (0,)), ((), ()))


_NH = 4
_CH = _B // _NH
_SPWC = _CH // _NW


def _sc_body(h, atoms_hbm, eloc_hbm, out_hbm,
             tv0, tv1, ev0, ev1, ac0, ac1, sin0, sin1, sout0, sout1):
    c = lax.axis_index("c")
    s = lax.axis_index("s")
    wid = s * 2 + c
    base = h * _CH + wid * _SPWC
    obase = wid * _SPWC
    tvs, evs, acs = (tv0, tv1), (ev0, ev1), (ac0, ac1)
    sins, souts = (sin0, sin1), (sout0, sout1)

    for tv in tvs:
        for j in range(_FC):
            tv[_A, pl.ds(j * 16, 16)] = jnp.zeros((16,), jnp.float32)

    def in_copies(g, bi):
        return (
            pltpu.make_async_copy(atoms_hbm.at[pl.ds(g * _A, _A)],
                                  tvs[bi].at[pl.ds(0, _A)], sins[bi]),
            pltpu.make_async_copy(eloc_hbm.at[g],
                                  evs[bi].at[pl.ds(0, _A * _D)], sins[bi]),
        )

    def out_copy(g, bi):
        return pltpu.make_async_copy(
            acs[bi], out_hbm.at[pl.ds((g - base + obase) * _A, _A)],
            souts[bi])

    for cp in in_copies(base, 0):
        cp.start()

    def pair_body(k, carry):
        for bi in range(2):
            g = base + k * 2 + bi
            for cp in in_copies(g, bi):
                cp.wait()

            @pl.when(g + 1 < base + _SPWC)
            def _():
                for cp in in_copies(g + 1, 1 - bi):
                    cp.start()

            @pl.when(g - 2 >= base)
            def _():
                out_copy(g - 2, bi).wait()

            tv, av = tvs[bi], acs[bi]

            def atom_body(a, carry2):
                evec = evs[bi][pl.ds(a * _D, 16)]
                es = [evec[d] for d in range(_D)]
                for j in range(_FC):
                    sl = pl.ds(j * 16, 16)
                    r = [tv[a, sl]] + [tv[e, sl] for e in es]
                    av[a, sl] = ((r[0] + r[1]) + (r[2] + r[3])) + \
                                ((r[4] + r[5]) + r[6])
                return carry2

            lax.fori_loop(0, _A, atom_body, 0, unroll=2)
            out_copy(g, bi).start()
        return carry

    lax.fori_loop(0, _SPWC // 2, pair_body, 0)
    out_copy(base + _SPWC - 2, 0).wait()
    out_copy(base + _SPWC - 1, 1).wait()


def _tc_body(et_ref, sa_ref, bt_ref, wa_ref, wb_ref, bias_ref,
             sel_ref, red_ref, out_ref):
    et = et_ref[...]
    sa16 = sa_ref[...].astype(jnp.bfloat16)
    bt16 = bt_ref[...].astype(jnp.bfloat16)

    valid16 = (et != -1).astype(jnp.bfloat16)
    ones_col = jnp.ones((_D, 1), jnp.bfloat16)
    deg = lax.dot_general(valid16, ones_col, _DN0,
                          preferred_element_type=jnp.float32)
    mask = (deg.astype(jnp.bfloat16) == sel_ref[...]).astype(jnp.bfloat16)

    y = jnp.dot(sa16, wa_ref[...], preferred_element_type=jnp.float32)
    y = y + lax.dot_general(bt16, wb_ref[...], _DN0,
                            preferred_element_type=jnp.float32)
    y = y + bias_ref[...]
    ym = y.astype(jnp.bfloat16) * mask
    out_ref[...] = jnp.dot(ym, red_ref[...], preferred_element_type=jnp.float32)


def kernel(atoms, bonds, edges, W, b):
    atoms2 = atoms.reshape(_B * _A, _F)
    eloc = jnp.where(edges >= 0, edges, _A).reshape(_B, _A * _D)
    et = edges.reshape(_B * _A, _D).T
    bt = bonds.reshape(_B * _A, _D * _FB).T
    wa = W[:, :_F, :].transpose(1, 0, 2).reshape(_F, _D * _CW
                                                 ).astype(jnp.bfloat16)
    wb = jnp.tile(W[:, _F:, :].transpose(1, 0, 2).reshape(_FB, _D * _CW),
                  (_D, 1)).astype(jnp.bfloat16)
    bias = b.reshape(1, _D * _CW)
    sel = jnp.asarray(np.repeat(np.arange(_D, dtype=np.float32), _CW)
                      ).reshape(1, _D * _CW).astype(jnp.bfloat16)
    red = jnp.asarray(
        (np.arange(_D * _CW)[:, None] % _CW == np.arange(_CW)[None, :])
        .astype(np.float32)).astype(jnp.bfloat16)

    import functools
    halves = []
    for h in range(_NH):
        sc_gather = pl.kernel(
            functools.partial(_sc_body, h),
            out_type=jax.ShapeDtypeStruct((_CH * _A, _F), jnp.float32),
            mesh=plsc.VectorSubcoreMesh(core_axis_name="c",
                                        subcore_axis_name="s"),
            scratch_types=[
                pltpu.VMEM((_A + 1, _F), jnp.float32),
                pltpu.VMEM((_A + 1, _F), jnp.float32),
                pltpu.VMEM((_A * 8 + 16,), jnp.int32),
                pltpu.VMEM((_A * 8 + 16,), jnp.int32),
                pltpu.VMEM((_A, _F), jnp.float32),
                pltpu.VMEM((_A, _F), jnp.float32),
                pltpu.SemaphoreType.DMA,
                pltpu.SemaphoreType.DMA,
                pltpu.SemaphoreType.DMA,
                pltpu.SemaphoreType.DMA,
            ],
            name=f"sc_gather_h{h}",
        )
        summed = sc_gather(atoms2, eloc)

        hb = _CH // _G
        out_h = pl.pallas_call(
            _tc_body,
            grid=(hb,),
            in_specs=[
                pl.BlockSpec((_D, _GA), lambda i, h=h: (0, i + h * hb)),
                pl.BlockSpec((_GA, _F), lambda i: (i, 0)),
                pl.BlockSpec((_D * _FB, _GA), lambda i, h=h: (0, i + h * hb)),
                pl.BlockSpec((_F, _D * _CW), lambda i: (0, 0)),
                pl.BlockSpec((_D * _FB, _D * _CW), lambda i: (0, 0)),
                pl.BlockSpec((1, _D * _CW), lambda i: (0, 0)),
                pl.BlockSpec((1, _D * _CW), lambda i: (0, 0)),
                pl.BlockSpec((_D * _CW, _CW), lambda i: (0, 0)),
            ],
            out_specs=pl.BlockSpec((_GA, _CW), lambda i: (i, 0)),
            out_shape=jax.ShapeDtypeStruct((_CH * _A, _CW), jnp.float32),
        )(et, summed, bt, wa, wb, bias, sel, red)
        halves.append(out_h)

    out = jnp.concatenate(halves, axis=0)
    return out.reshape(_B, _A, _CW)
